# baseline (speedup 1.0000x reference)
import functools

import jax
import jax.numpy as jnp
import numpy as np
from jax import lax
from jax.experimental import pallas as pl
from jax.experimental.pallas import tpu as pltpu
from jax.experimental.pallas import tpu_sc as plsc

VOCAB = 50001
D_MODEL = 128
B = 4
L = 2048

NC = 2
NS = 16
NW = NC * NS
N_ROWS = B * L
CHUNK = L // NW
N_LANE_SL = D_MODEL // 16


def _pos_encoding(seq_len, d_model):
    pos = jnp.arange(seq_len, dtype=jnp.float32)[:, None]
    div = jnp.exp(jnp.arange(0, d_model, 2, dtype=jnp.float32)
                  * (-np.log(10000.0) / d_model))
    pe = jnp.zeros((seq_len, d_model), dtype=jnp.float32)
    pe = pe.at[:, 0::2].set(jnp.sin(pos * div))
    pe = pe.at[:, 1::2].set(jnp.cos(pos * div))
    return pe


@functools.partial(
    pl.kernel,
    out_type=jax.ShapeDtypeStruct((N_ROWS, D_MODEL), jnp.float32),
    mesh=plsc.VectorSubcoreMesh(core_axis_name="c", subcore_axis_name="s"),
    scratch_types=[
        pltpu.VMEM((B, CHUNK), jnp.int32),
        pltpu.VMEM((B * CHUNK, D_MODEL), jnp.float32),
        pltpu.VMEM_SHARED((L, D_MODEL), jnp.float32),
        pltpu.VMEM((CHUNK,), jnp.int32),
        pltpu.SemaphoreType.DMA((B,)),
        pltpu.SemaphoreType.DMA,
        pltpu.SemaphoreType.DMA((B,)),
        pltpu.SemaphoreType.DMA((B,)),
        pltpu.SemaphoreType.DMA((B,)),
    ],
)
def _sc_embed(x_hbm, pe_hbm, table_hbm, out_hbm, idx_v, rows_v, pe_sh,
              prow_v, isems, psem, gsems, asems, ssems):
    s = lax.axis_index("s")
    w = s * NC + lax.axis_index("c")
    col = w * CHUNK
    for t in range(CHUNK // 16):
        prow_v[pl.ds(t * 16, 16)] = lax.iota(jnp.int32, 16) + (col + t * 16)
    idx_cps = [
        pltpu.async_copy(x_hbm.at[j, pl.ds(col, CHUNK)], idx_v.at[j],
                         isems.at[j])
        for j in range(B)
    ]
    @pl.when(s == 0)
    def _():
        pltpu.sync_copy(pe_hbm, pe_sh)

    g_cps = []
    for j in range(B):
        idx_cps[j].wait()
        g_cps.append(
            pltpu.async_copy(table_hbm.at[idx_v.at[j]],
                             rows_v.at[pl.ds(j * CHUNK, CHUNK)],
                             gsems.at[j]))
    plsc.subcore_barrier()
    a_cps = []
    for j in range(B):
        g_cps[j].wait()
        a_cps.append(
            pltpu.async_copy(pe_sh.at[prow_v],
                             rows_v.at[pl.ds(j * CHUNK, CHUNK)],
                             asems.at[j], add=True))
    s_cps = []
    for j in range(B):
        a_cps[j].wait()
        s_cps.append(
            pltpu.async_copy(rows_v.at[pl.ds(j * CHUNK, CHUNK)],
                             out_hbm.at[pl.ds(j * L + col, CHUNK)],
                             ssems.at[j]))
    for cp in s_cps:
        cp.wait()


def kernel(x, tgt, emb_src, emb_tgt):
    del tgt, emb_tgt
    pe = _pos_encoding(L, D_MODEL)
    out = _sc_embed(x, pe, emb_src)
    return out.reshape(B, L, D_MODEL)


# speedup vs baseline: 1.0289x; 1.0289x over previous
import functools

import jax
import jax.numpy as jnp
import numpy as np
from jax import lax
from jax.experimental import pallas as pl
from jax.experimental.pallas import tpu as pltpu
from jax.experimental.pallas import tpu_sc as plsc

VOCAB = 50001
D_MODEL = 128
B = 4
L = 2048

NC = 2
NS = 16
NW = NC * NS
N_ROWS = B * L
CHUNK = L // NW
N_LANE_SL = D_MODEL // 16


def _pos_encoding(seq_len, d_model):
    pos = jnp.arange(seq_len, dtype=jnp.float32)[:, None]
    div = jnp.exp(jnp.arange(0, d_model, 2, dtype=jnp.float32)
                  * (-np.log(10000.0) / d_model))
    pe = jnp.zeros((seq_len, d_model), dtype=jnp.float32)
    pe = pe.at[:, 0::2].set(jnp.sin(pos * div))
    pe = pe.at[:, 1::2].set(jnp.cos(pos * div))
    return pe


@functools.partial(
    pl.kernel,
    out_type=jax.ShapeDtypeStruct((N_ROWS, D_MODEL), jnp.float32),
    mesh=plsc.VectorSubcoreMesh(core_axis_name="c", subcore_axis_name="s"),
    scratch_types=[
        pltpu.VMEM((B // 2, 2 * CHUNK), jnp.int32),
        pltpu.VMEM((B * CHUNK, D_MODEL), jnp.float32),
        pltpu.VMEM((CHUNK, D_MODEL), jnp.float32),
        pltpu.SemaphoreType.DMA((B,)),
        pltpu.SemaphoreType.DMA,
        pltpu.SemaphoreType.DMA((B // 2,)),
        pltpu.SemaphoreType.DMA((B,)),
    ],
)
def _sc_embed(x_hbm, pe_hbm, table_hbm, out_hbm, idx_v, rows_v, pe_v,
              isems, psem, gsems, ssems):
    w = lax.axis_index("s") * NC + lax.axis_index("c")
    col = w * CHUNK
    idx_cps = [
        pltpu.async_copy(x_hbm.at[j, pl.ds(col, CHUNK)],
                         idx_v.at[j // 2, pl.ds((j % 2) * CHUNK, CHUNK)],
                         isems.at[j])
        for j in range(B)
    ]
    pe_cp = pltpu.async_copy(pe_hbm.at[pl.ds(col, CHUNK)], pe_v, psem)
    g_cps = []
    for jj in range(B // 2):
        idx_cps[2 * jj].wait()
        idx_cps[2 * jj + 1].wait()
        g_cps.append(
            pltpu.async_copy(table_hbm.at[idx_v.at[jj]],
                             rows_v.at[pl.ds(jj * 2 * CHUNK, 2 * CHUNK)],
                             gsems.at[jj]))
    pe_cp.wait()
    s_cps = []
    for jj in range(B // 2):
        g_cps[jj].wait()

        def add_row(r, carry, jj=jj):
            for t in range(N_LANE_SL):
                sl = pl.ds(t * 16, 16)
                pv = pe_v[r, sl]
                rows_v[jj * 2 * CHUNK + r, sl] = (
                    rows_v[jj * 2 * CHUNK + r, sl] + pv)
                rows_v[jj * 2 * CHUNK + CHUNK + r, sl] = (
                    rows_v[jj * 2 * CHUNK + CHUNK + r, sl] + pv)
            return carry

        lax.fori_loop(0, CHUNK, add_row, 0)
        for h in range(2):
            s_cps.append(
                pltpu.async_copy(
                    rows_v.at[pl.ds((jj * 2 + h) * CHUNK, CHUNK)],
                    out_hbm.at[pl.ds((2 * jj + h) * L + col, CHUNK)],
                    ssems.at[2 * jj + h]))
    for cp in s_cps:
        cp.wait()


def kernel(x, tgt, emb_src, emb_tgt):
    del tgt, emb_tgt
    pe = _pos_encoding(L, D_MODEL)
    out = _sc_embed(x, pe, emb_src)
    return out.reshape(B, L, D_MODEL)
